# Initial kernel scaffold; baseline (speedup 1.0000x reference)
import jax
import jax.numpy as jnp
from jax.experimental import pallas as pl


def kernel(x, edge_index, W0, g0, b0, W1, g1, b1, W2, g2, b2, Wc, bc):
    raise NotImplementedError("write your pallas kernel here")


# speedup vs baseline: 6.0361x; 6.0361x over previous
import functools

import jax
import jax.numpy as jnp
from jax import lax
from jax.experimental import pallas as pl
from jax.experimental.pallas import tpu as pltpu
from jax.experimental.pallas import tpu_sc as plsc

N = 10000
NP = 10240
E = 320000
D = 128
NC = 2
NS = 16
CH = 80
ET = E // NS
TCHUNKS = ET // CH
ET2 = E // (NC * NS)
TCHUNKS2 = ET2 // CH
RPT = NP // NS

_MESH = plsc.VectorSubcoreMesh(
    core_axis_name="c", subcore_axis_name="s", num_cores=NC, num_subcores=NS)


@functools.partial(
    pl.kernel,
    out_type=jax.ShapeDtypeStruct((NC * NP, D), jnp.float32),
    mesh=_MESH,
    scratch_types=[
        pltpu.VMEM((ET,), jnp.int32),
        pltpu.VMEM((CH,), jnp.int32),
        pltpu.VMEM((CH, D), jnp.float32),
        pltpu.VMEM_SHARED((NP, D), jnp.float32),
    ],
)
def _deg_kernel(eidx_hbm, zeros_hbm, ones_hbm, out_hbm, idx_v, didx_v, ones_v, deg_sh):
    c = lax.axis_index("c")
    s = lax.axis_index("s")

    pltpu.sync_copy(ones_hbm, ones_v)

    base = pl.multiple_of(c * E + s * ET, 8)
    pltpu.sync_copy(eidx_hbm.at[pl.ds(base, ET)], idx_v)

    pltpu.sync_copy(zeros_hbm.at[pl.ds(s * RPT, RPT)], deg_sh.at[pl.ds(s * RPT, RPT)])
    plsc.subcore_barrier()

    def step(j, _):
        for i in range(CH // 16):
            didx_v[pl.ds(i * 16, 16)] = idx_v[pl.ds(j * CH + i * 16, 16)]
        pltpu.sync_copy(ones_v, deg_sh.at[didx_v], add=True)
        return 0
    lax.fori_loop(0, TCHUNKS, step, 0)

    plsc.subcore_barrier()
    obase = pl.multiple_of(c * NP + s * RPT, 8)
    pltpu.sync_copy(deg_sh.at[pl.ds(s * RPT, RPT)], out_hbm.at[pl.ds(obase, RPT)])


@functools.partial(
    pl.kernel,
    out_type=jax.ShapeDtypeStruct((NC * NP, D), jnp.float32),
    mesh=_MESH,
    scratch_types=[
        pltpu.VMEM((ET2,), jnp.int32),
        pltpu.VMEM((ET2,), jnp.int32),
        pltpu.VMEM((CH,), jnp.int32),
        pltpu.VMEM((CH,), jnp.int32),
        pltpu.VMEM((CH, D), jnp.float32),
        pltpu.VMEM_SHARED((NP, D), jnp.float32),
    ],
)
def _scatter_kernel(hw_hbm, eidx_hbm, zeros_hbm, out_hbm,
                    src_v, dst_v, gidx_v, didx_v, rows_v, agg_sh):
    c = lax.axis_index("c")
    s = lax.axis_index("s")

    sbase = pl.multiple_of((c * NS + s) * ET2, 8)
    dbase = pl.multiple_of(E + (c * NS + s) * ET2, 8)
    pltpu.sync_copy(eidx_hbm.at[pl.ds(sbase, ET2)], src_v)
    pltpu.sync_copy(eidx_hbm.at[pl.ds(dbase, ET2)], dst_v)
    pltpu.sync_copy(zeros_hbm.at[pl.ds(s * RPT, RPT)], agg_sh.at[pl.ds(s * RPT, RPT)])
    plsc.subcore_barrier()

    def step(j, _):
        for i in range(CH // 16):
            sl = pl.ds(i * 16, 16)
            gidx_v[sl] = src_v[pl.ds(j * CH + i * 16, 16)]
            didx_v[sl] = dst_v[pl.ds(j * CH + i * 16, 16)]
        pltpu.sync_copy(hw_hbm.at[gidx_v], rows_v)
        pltpu.sync_copy(rows_v, agg_sh.at[didx_v], add=True)
        return 0
    lax.fori_loop(0, TCHUNKS2, step, 0)

    plsc.subcore_barrier()
    obase = pl.multiple_of(c * NP + s * RPT, 8)
    pltpu.sync_copy(agg_sh.at[pl.ds(s * RPT, RPT)], out_hbm.at[pl.ds(obase, RPT)])


def _scales(degs):
    s_out = lax.rsqrt(jnp.maximum(degs[0:N, 0:1], 1.0))
    s_in = lax.rsqrt(jnp.maximum(degs[NP:NP + N, 0:1], 1.0))
    return s_out, s_in


def _f0_body(x_ref, degs_ref, w_ref, out_ref):
    s_out, _ = _scales(degs_ref[...])
    out_ref[0:N, :] = lax.dot_general(x_ref[...] * s_out, w_ref[...],
                                      (((1,), (1,)), ((), ())),
                                      preferred_element_type=jnp.float32)


def _bn_relu(agg_ref, degs_ref, g_ref, b_ref):
    a = agg_ref[...]
    h = a[0:N] + a[NP:NP + N]
    s_out, s_in = _scales(degs_ref[...])
    h = h * s_in
    m = jnp.mean(h, axis=0, keepdims=True)
    d = h - m
    v = jnp.mean(d * d, axis=0, keepdims=True)
    hn = d * lax.rsqrt(v + 1e-5) * g_ref[...] + b_ref[...]
    return jnp.maximum(hn, 0.0), s_out


def _f1_body(agg_ref, degs_ref, g_ref, b_ref, w_ref, out_ref):
    hr, s_out = _bn_relu(agg_ref, degs_ref, g_ref, b_ref)
    out_ref[0:N, :] = lax.dot_general(hr * s_out, w_ref[...],
                                      (((1,), (1,)), ((), ())),
                                      preferred_element_type=jnp.float32)


def _fc_body(agg_ref, degs_ref, g_ref, b_ref, wc_ref, bc_ref, out_ref):
    hr, _ = _bn_relu(agg_ref, degs_ref, g_ref, b_ref)
    out_ref[...] = lax.dot_general(hr, wc_ref[...],
                                   (((1,), (1,)), ((), ())),
                                   preferred_element_type=jnp.float32) + bc_ref[...]


_f0 = pl.pallas_call(_f0_body, out_shape=jax.ShapeDtypeStruct((NP, D), jnp.float32))
_f1 = pl.pallas_call(_f1_body, out_shape=jax.ShapeDtypeStruct((NP, D), jnp.float32))
_fc = pl.pallas_call(_fc_body, out_shape=jax.ShapeDtypeStruct((N, 40), jnp.float32))


def kernel(x, edge_index, W0, g0, b0, W1, g1, b1, W2, g2, b2, Wc, bc):
    eidx = edge_index.reshape(2 * E)
    zeros128 = jnp.zeros((NP, D), jnp.float32)
    g0r, g1r, g2r = g0.reshape(1, D), g1.reshape(1, D), g2.reshape(1, D)
    b0r, b1r, b2r = b0.reshape(1, D), b1.reshape(1, D), b2.reshape(1, D)
    bcr = bc.reshape(1, 40)

    degs = _deg_kernel(eidx, zeros128, jnp.ones((CH, D), jnp.float32))
    hw = _f0(x, degs, W0)
    agg = _scatter_kernel(hw, eidx, zeros128)
    hw = _f1(agg, degs, g0r, b0r, W1)
    agg = _scatter_kernel(hw, eidx, zeros128)
    hw = _f1(agg, degs, g1r, b1r, W2)
    agg = _scatter_kernel(hw, eidx, zeros128)
    return _fc(agg, degs, g2r, b2r, Wc, bcr)


# speedup vs baseline: 8.5266x; 1.4126x over previous
import functools

import jax
import jax.numpy as jnp
from jax import lax
from jax.experimental import pallas as pl
from jax.experimental.pallas import tpu as pltpu
from jax.experimental.pallas import tpu_sc as plsc

N = 10000
NP = 10240
E = 320000
D = 128
NC = 2
NS = 16
CH = 80
ET = E // NS
TCHUNKS = ET // CH
ET2 = E // (NC * NS)
TCHUNKS2 = ET2 // CH
RPT = NP // NS

_MESH = plsc.VectorSubcoreMesh(
    core_axis_name="c", subcore_axis_name="s", num_cores=NC, num_subcores=NS)


@functools.partial(
    pl.kernel,
    out_type=jax.ShapeDtypeStruct((NC * NP,), jnp.float32),
    mesh=_MESH,
    scratch_types=[
        pltpu.VMEM((ET,), jnp.int32),
        pltpu.VMEM((2, CH), jnp.int32),
        pltpu.VMEM((CH,), jnp.float32),
        pltpu.VMEM_SHARED((NP,), jnp.float32),
        pltpu.SemaphoreType.DMA((2,)),
    ],
)
def _deg_kernel(eidx_hbm, zeros_hbm, ones_hbm, out_hbm,
                idx_v, didx_v, ones_v, deg_sh, sem_s):
    c = lax.axis_index("c")
    s = lax.axis_index("s")

    pltpu.sync_copy(ones_hbm, ones_v)
    base = pl.multiple_of(c * E + s * ET, 8)
    pltpu.sync_copy(eidx_hbm.at[pl.ds(base, ET)], idx_v)
    pltpu.sync_copy(zeros_hbm.at[pl.ds(s * RPT, RPT)], deg_sh.at[pl.ds(s * RPT, RPT)])
    plsc.subcore_barrier()

    def step(j, _):
        p = lax.rem(j, 2)
        q = 1 - p

        @pl.when(j > 0)
        def _():
            pltpu.make_async_copy(ones_v, deg_sh.at[didx_v.at[q]], sem_s.at[q]).wait()

        for i in range(CH // 16):
            didx_v[p, pl.ds(i * 16, 16)] = idx_v[pl.ds(j * CH + i * 16, 16)]
        pltpu.async_copy(ones_v, deg_sh.at[didx_v.at[p]], sem_s.at[p], add=True)
        return 0
    lax.fori_loop(0, TCHUNKS, step, 0)
    lastp = (TCHUNKS - 1) % 2
    pltpu.make_async_copy(ones_v, deg_sh.at[didx_v.at[lastp]], sem_s.at[lastp]).wait()

    plsc.subcore_barrier()
    obase = pl.multiple_of(c * NP + s * RPT, 8)
    pltpu.sync_copy(deg_sh.at[pl.ds(s * RPT, RPT)], out_hbm.at[pl.ds(obase, RPT)])


@functools.partial(
    pl.kernel,
    out_type=jax.ShapeDtypeStruct((NC * NP, D), jnp.float32),
    mesh=_MESH,
    scratch_types=[
        pltpu.VMEM((ET2,), jnp.int32),
        pltpu.VMEM((ET2,), jnp.int32),
        pltpu.VMEM((2, CH), jnp.int32),
        pltpu.VMEM((2, CH), jnp.int32),
        pltpu.VMEM((2, CH, D), jnp.float32),
        pltpu.VMEM_SHARED((NP, D), jnp.float32),
        pltpu.SemaphoreType.DMA((2,)),
        pltpu.SemaphoreType.DMA((2,)),
    ],
)
def _scatter_kernel(hw_hbm, eidx_hbm, zeros_hbm, out_hbm,
                    src_v, dst_v, gidx_v, didx_v, rows_v, agg_sh, sem_g, sem_s):
    c = lax.axis_index("c")
    s = lax.axis_index("s")

    sbase = pl.multiple_of((c * NS + s) * ET2, 8)
    dbase = pl.multiple_of(E + (c * NS + s) * ET2, 8)
    pltpu.sync_copy(eidx_hbm.at[pl.ds(sbase, ET2)], src_v)
    pltpu.sync_copy(eidx_hbm.at[pl.ds(dbase, ET2)], dst_v)
    pltpu.sync_copy(zeros_hbm.at[pl.ds(s * RPT, RPT)], agg_sh.at[pl.ds(s * RPT, RPT)])
    plsc.subcore_barrier()

    def fill_idx(j, p):
        for i in range(CH // 16):
            gidx_v[p, pl.ds(i * 16, 16)] = src_v[pl.ds(j * CH + i * 16, 16)]
            didx_v[p, pl.ds(i * 16, 16)] = dst_v[pl.ds(j * CH + i * 16, 16)]

    fill_idx(0, 0)
    pltpu.async_copy(hw_hbm.at[gidx_v.at[0]], rows_v.at[0], sem_g.at[0])

    def step(j, _):
        p = lax.rem(j, 2)
        q = 1 - p

        pltpu.make_async_copy(hw_hbm.at[gidx_v.at[p]], rows_v.at[p], sem_g.at[p]).wait()

        @pl.when(j > 0)
        def _():
            pltpu.make_async_copy(rows_v.at[q], agg_sh.at[didx_v.at[q]], sem_s.at[q]).wait()

        pltpu.async_copy(rows_v.at[p], agg_sh.at[didx_v.at[p]], sem_s.at[p], add=True)

        @pl.when(j + 1 < TCHUNKS2)
        def _():
            fill_idx(j + 1, q)
            pltpu.async_copy(hw_hbm.at[gidx_v.at[q]], rows_v.at[q], sem_g.at[q])
        return 0
    lax.fori_loop(0, TCHUNKS2, step, 0)
    lastp = (TCHUNKS2 - 1) % 2
    pltpu.make_async_copy(rows_v.at[lastp], agg_sh.at[didx_v.at[lastp]], sem_s.at[lastp]).wait()

    plsc.subcore_barrier()
    obase = pl.multiple_of(c * NP + s * RPT, 8)
    pltpu.sync_copy(agg_sh.at[pl.ds(s * RPT, RPT)], out_hbm.at[pl.ds(obase, RPT)])


def _scale_of(deg_ref):
    return lax.rsqrt(jnp.maximum(deg_ref[...], 1.0))


def _f0_body(x_ref, do_ref, w_ref, out_ref):
    out_ref[0:N, :] = lax.dot_general(x_ref[...] * _scale_of(do_ref), w_ref[...],
                                      (((1,), (1,)), ((), ())),
                                      preferred_element_type=jnp.float32)


def _bn_relu(agg_ref, di_ref, g_ref, b_ref):
    a = agg_ref[...]
    h = a[0:N] + a[NP:NP + N]
    h = h * _scale_of(di_ref)
    m = jnp.mean(h, axis=0, keepdims=True)
    d = h - m
    v = jnp.mean(d * d, axis=0, keepdims=True)
    hn = d * lax.rsqrt(v + 1e-5) * g_ref[...] + b_ref[...]
    return jnp.maximum(hn, 0.0)


def _f1_body(agg_ref, do_ref, di_ref, g_ref, b_ref, w_ref, out_ref):
    hr = _bn_relu(agg_ref, di_ref, g_ref, b_ref)
    out_ref[0:N, :] = lax.dot_general(hr * _scale_of(do_ref), w_ref[...],
                                      (((1,), (1,)), ((), ())),
                                      preferred_element_type=jnp.float32)


def _fc_body(agg_ref, di_ref, g_ref, b_ref, wc_ref, bc_ref, out_ref):
    hr = _bn_relu(agg_ref, di_ref, g_ref, b_ref)
    out_ref[...] = lax.dot_general(hr, wc_ref[...],
                                   (((1,), (1,)), ((), ())),
                                   preferred_element_type=jnp.float32) + bc_ref[...]


_f0 = pl.pallas_call(_f0_body, out_shape=jax.ShapeDtypeStruct((NP, D), jnp.float32))
_f1 = pl.pallas_call(_f1_body, out_shape=jax.ShapeDtypeStruct((NP, D), jnp.float32))
_fc = pl.pallas_call(_fc_body, out_shape=jax.ShapeDtypeStruct((N, 40), jnp.float32))


def kernel(x, edge_index, W0, g0, b0, W1, g1, b1, W2, g2, b2, Wc, bc):
    eidx = edge_index.reshape(2 * E)
    zeros1 = jnp.zeros((NP,), jnp.float32)
    zeros128 = jnp.zeros((NP, D), jnp.float32)
    g0r, g1r, g2r = g0.reshape(1, D), g1.reshape(1, D), g2.reshape(1, D)
    b0r, b1r, b2r = b0.reshape(1, D), b1.reshape(1, D), b2.reshape(1, D)
    bcr = bc.reshape(1, 40)

    degs = _deg_kernel(eidx, zeros1, jnp.ones((CH,), jnp.float32))
    do = degs[0:N].reshape(N, 1)
    di = degs[NP:NP + N].reshape(N, 1)

    hw = _f0(x, do, W0)
    agg = _scatter_kernel(hw, eidx, zeros128)
    hw = _f1(agg, do, di, g0r, b0r, W1)
    agg = _scatter_kernel(hw, eidx, zeros128)
    hw = _f1(agg, do, di, g1r, b1r, W2)
    agg = _scatter_kernel(hw, eidx, zeros128)
    return _fc(agg, di, g2r, b2r, Wc, bcr)


# speedup vs baseline: 12.3370x; 1.4469x over previous
import functools

import jax
import jax.numpy as jnp
from jax import lax
from jax.experimental import pallas as pl
from jax.experimental.pallas import tpu as pltpu
from jax.experimental.pallas import tpu_sc as plsc

N = 10000
NP = 10240
E = 320000
D = 128
NC = 2
NS = 16
CH = 80
ET = E // NS
TCHUNKS = ET // CH
ET2 = E // (NC * NS)
TCHUNKS2 = ET2 // CH
RPT = NP // NS

_MESH = plsc.VectorSubcoreMesh(
    core_axis_name="c", subcore_axis_name="s", num_cores=NC, num_subcores=NS)


@functools.partial(
    pl.kernel,
    out_type=jax.ShapeDtypeStruct((NC * NP,), jnp.float32),
    mesh=_MESH,
    scratch_types=[
        pltpu.VMEM((ET,), jnp.int32),
        pltpu.VMEM((2, CH), jnp.int32),
        pltpu.VMEM((CH,), jnp.float32),
        pltpu.VMEM_SHARED((NP,), jnp.float32),
        pltpu.SemaphoreType.DMA((2,)),
    ],
)
def _deg_kernel(eidx_hbm, zeros_hbm, ones_hbm, out_hbm,
                idx_v, didx_v, ones_v, deg_sh, sem_s):
    c = lax.axis_index("c")
    s = lax.axis_index("s")

    pltpu.sync_copy(ones_hbm, ones_v)
    base = pl.multiple_of(c * E + s * ET, 8)
    pltpu.sync_copy(eidx_hbm.at[pl.ds(base, ET)], idx_v)
    pltpu.sync_copy(zeros_hbm.at[pl.ds(s * RPT, RPT)], deg_sh.at[pl.ds(s * RPT, RPT)])
    plsc.subcore_barrier()

    def step(j, _):
        p = lax.rem(j, 2)
        q = 1 - p

        @pl.when(j > 0)
        def _():
            pltpu.make_async_copy(ones_v, deg_sh.at[didx_v.at[q]], sem_s.at[q]).wait()

        for i in range(CH // 16):
            didx_v[p, pl.ds(i * 16, 16)] = idx_v[pl.ds(j * CH + i * 16, 16)]
        pltpu.async_copy(ones_v, deg_sh.at[didx_v.at[p]], sem_s.at[p], add=True)
        return 0
    lax.fori_loop(0, TCHUNKS, step, 0)
    lastp = (TCHUNKS - 1) % 2
    pltpu.make_async_copy(ones_v, deg_sh.at[didx_v.at[lastp]], sem_s.at[lastp]).wait()

    plsc.subcore_barrier()
    obase = pl.multiple_of(c * NP + s * RPT, 8)
    pltpu.sync_copy(deg_sh.at[pl.ds(s * RPT, RPT)], out_hbm.at[pl.ds(obase, RPT)])


NBUF = 4


@functools.partial(
    pl.kernel,
    out_type=jax.ShapeDtypeStruct((NC * NP, D), jnp.float32),
    mesh=_MESH,
    scratch_types=[
        pltpu.VMEM((NBUF, CH), jnp.int32),
        pltpu.VMEM((NBUF, CH), jnp.int32),
        pltpu.VMEM((NBUF, CH, D), jnp.float32),
        pltpu.VMEM_SHARED((NP, D), jnp.float32),
        pltpu.SemaphoreType.DMA((NBUF,)),
        pltpu.SemaphoreType.DMA((NBUF,)),
        pltpu.SemaphoreType.DMA((NBUF,)),
    ],
)
def _scatter_kernel(hw_hbm, eidx_hbm, zeros_hbm, out_hbm,
                    sidx_v, didx_v, rows_v, agg_sh, sem_i, sem_g, sem_s):
    c = lax.axis_index("c")
    s = lax.axis_index("s")
    tbase = (c * NS + s) * ET2

    def idx_refs(j, sl):
        sb = pl.ds(pl.multiple_of(tbase + j * CH, 8), CH)
        db = pl.ds(pl.multiple_of(E + tbase + j * CH, 8), CH)
        return ((eidx_hbm.at[sb], sidx_v.at[sl], sem_i.at[sl]),
                (eidx_hbm.at[db], didx_v.at[sl], sem_i.at[sl]))

    def idx_start(j, sl):
        for tr in idx_refs(j, sl):
            pltpu.async_copy(*tr)

    def idx_wait(j, sl):
        for tr in idx_refs(j, sl):
            pltpu.make_async_copy(*tr).wait()

    def gather_refs(sl):
        return (hw_hbm.at[sidx_v.at[sl]], rows_v.at[sl], sem_g.at[sl])

    def scat_refs(sl):
        return (rows_v.at[sl], agg_sh.at[didx_v.at[sl]], sem_s.at[sl])

    pltpu.sync_copy(zeros_hbm.at[pl.ds(s * RPT, RPT)], agg_sh.at[pl.ds(s * RPT, RPT)])
    plsc.subcore_barrier()

    for jj in range(min(NBUF - 1, TCHUNKS2)):
        idx_start(jj, jj)
    for jj in range(min(2, TCHUNKS2)):
        idx_wait(jj, jj)
        pltpu.async_copy(*gather_refs(jj))

    def step(j, _):
        p = lax.rem(j, NBUF)
        q = lax.rem(j + NBUF - 1, NBUF)

        pltpu.make_async_copy(*gather_refs(p)).wait()

        @pl.when(j > 0)
        def _():
            pltpu.make_async_copy(*scat_refs(q)).wait()

        pltpu.async_copy(*scat_refs(p), add=True)

        @pl.when(j + NBUF - 1 < TCHUNKS2)
        def _():
            idx_start(j + NBUF - 1, q)

        @pl.when(j + 2 < TCHUNKS2)
        def _():
            r = lax.rem(j + 2, NBUF)
            idx_wait(j + 2, r)
            pltpu.async_copy(*gather_refs(r))
        return 0
    lax.fori_loop(0, TCHUNKS2, step, 0)
    lastp = (TCHUNKS2 - 1) % NBUF
    pltpu.make_async_copy(*scat_refs(lastp)).wait()

    plsc.subcore_barrier()
    obase = pl.multiple_of(c * NP + s * RPT, 8)
    pltpu.sync_copy(agg_sh.at[pl.ds(s * RPT, RPT)], out_hbm.at[pl.ds(obase, RPT)])


def _scale_of(deg_ref):
    return lax.rsqrt(jnp.maximum(deg_ref[...], 1.0))


def _f0_body(x_ref, do_ref, w_ref, out_ref):
    out_ref[0:N, :] = lax.dot_general(x_ref[...] * _scale_of(do_ref), w_ref[...],
                                      (((1,), (1,)), ((), ())),
                                      preferred_element_type=jnp.float32)


def _bn_relu(agg_ref, di_ref, g_ref, b_ref):
    a = agg_ref[...]
    h = a[0:N] + a[NP:NP + N]
    h = h * _scale_of(di_ref)
    m = jnp.mean(h, axis=0, keepdims=True)
    d = h - m
    v = jnp.mean(d * d, axis=0, keepdims=True)
    hn = d * lax.rsqrt(v + 1e-5) * g_ref[...] + b_ref[...]
    return jnp.maximum(hn, 0.0)


def _f1_body(agg_ref, do_ref, di_ref, g_ref, b_ref, w_ref, out_ref):
    hr = _bn_relu(agg_ref, di_ref, g_ref, b_ref)
    out_ref[0:N, :] = lax.dot_general(hr * _scale_of(do_ref), w_ref[...],
                                      (((1,), (1,)), ((), ())),
                                      preferred_element_type=jnp.float32)


def _fc_body(agg_ref, di_ref, g_ref, b_ref, wc_ref, bc_ref, out_ref):
    hr = _bn_relu(agg_ref, di_ref, g_ref, b_ref)
    out_ref[...] = lax.dot_general(hr, wc_ref[...],
                                   (((1,), (1,)), ((), ())),
                                   preferred_element_type=jnp.float32) + bc_ref[...]


_f0 = pl.pallas_call(_f0_body, out_shape=jax.ShapeDtypeStruct((NP, D), jnp.float32))
_f1 = pl.pallas_call(_f1_body, out_shape=jax.ShapeDtypeStruct((NP, D), jnp.float32))
_fc = pl.pallas_call(_fc_body, out_shape=jax.ShapeDtypeStruct((N, 40), jnp.float32))


def kernel(x, edge_index, W0, g0, b0, W1, g1, b1, W2, g2, b2, Wc, bc):
    eidx = edge_index.reshape(2 * E)
    zeros1 = jnp.zeros((NP,), jnp.float32)
    zeros128 = jnp.zeros((NP, D), jnp.float32)
    g0r, g1r, g2r = g0.reshape(1, D), g1.reshape(1, D), g2.reshape(1, D)
    b0r, b1r, b2r = b0.reshape(1, D), b1.reshape(1, D), b2.reshape(1, D)
    bcr = bc.reshape(1, 40)

    degs = _deg_kernel(eidx, zeros1, jnp.ones((CH,), jnp.float32))
    do = degs[0:N].reshape(N, 1)
    di = degs[NP:NP + N].reshape(N, 1)

    hw = _f0(x, do, W0)
    agg = _scatter_kernel(hw, eidx, zeros128)
    hw = _f1(agg, do, di, g0r, b0r, W1)
    agg = _scatter_kernel(hw, eidx, zeros128)
    hw = _f1(agg, do, di, g1r, b1r, W2)
    agg = _scatter_kernel(hw, eidx, zeros128)
    return _fc(agg, di, g2r, b2r, Wc, bcr)


# speedup vs baseline: 12.7059x; 1.0299x over previous
import functools

import jax
import jax.numpy as jnp
from jax import lax
from jax.experimental import pallas as pl
from jax.experimental.pallas import tpu as pltpu
from jax.experimental.pallas import tpu_sc as plsc

N = 10000
NP = 10240
E = 320000
D = 128
NC = 2
NS = 16
CH = 80
ET = E // NS
TCHUNKS = ET // CH
ET2 = E // (NC * NS)
TCHUNKS2 = ET2 // CH
RPT = NP // NS

_MESH = plsc.VectorSubcoreMesh(
    core_axis_name="c", subcore_axis_name="s", num_cores=NC, num_subcores=NS)


@functools.partial(
    pl.kernel,
    out_type=jax.ShapeDtypeStruct((NC * NP,), jnp.float32),
    mesh=_MESH,
    scratch_types=[
        pltpu.VMEM((ET,), jnp.int32),
        pltpu.VMEM((3, CH), jnp.int32),
        pltpu.VMEM((CH,), jnp.float32),
        pltpu.VMEM_SHARED((NP,), jnp.float32),
        pltpu.SemaphoreType.DMA((3,)),
    ],
)
def _deg_kernel(eidx_hbm, zeros_hbm, ones_hbm, out_hbm,
                idx_v, didx_v, ones_v, deg_sh, sem_s):
    c = lax.axis_index("c")
    s = lax.axis_index("s")

    pltpu.sync_copy(ones_hbm, ones_v)
    base = pl.multiple_of(c * E + s * ET, 8)
    pltpu.sync_copy(eidx_hbm.at[pl.ds(base, ET)], idx_v)
    pltpu.sync_copy(zeros_hbm.at[pl.ds(s * RPT, RPT)], deg_sh.at[pl.ds(s * RPT, RPT)])
    plsc.subcore_barrier()

    def step(j, _):
        p = lax.rem(j, 3)

        @pl.when(j > 1)
        def _():
            q = lax.rem(j + 1, 3)
            pltpu.make_async_copy(ones_v, deg_sh.at[didx_v.at[q]], sem_s.at[q]).wait()

        for i in range(CH // 16):
            didx_v[p, pl.ds(i * 16, 16)] = idx_v[pl.ds(j * CH + i * 16, 16)]
        pltpu.async_copy(ones_v, deg_sh.at[didx_v.at[p]], sem_s.at[p], add=True)
        return 0
    lax.fori_loop(0, TCHUNKS, step, 0)
    for jj in (TCHUNKS - 2, TCHUNKS - 1):
        pltpu.make_async_copy(ones_v, deg_sh.at[didx_v.at[jj % 3]],
                              sem_s.at[jj % 3]).wait()

    plsc.subcore_barrier()
    obase = pl.multiple_of(c * NP + s * RPT, 8)
    pltpu.sync_copy(deg_sh.at[pl.ds(s * RPT, RPT)], out_hbm.at[pl.ds(obase, RPT)])


NROWS = 4
NIDX = 5


@functools.partial(
    pl.kernel,
    out_type=jax.ShapeDtypeStruct((NC * NP, D), jnp.float32),
    mesh=_MESH,
    scratch_types=[
        pltpu.VMEM((NIDX, CH), jnp.int32),
        pltpu.VMEM((NIDX, CH), jnp.int32),
        pltpu.VMEM((NROWS, CH, D), jnp.float32),
        pltpu.VMEM_SHARED((NP, D), jnp.float32),
        pltpu.SemaphoreType.DMA((NIDX,)),
        pltpu.SemaphoreType.DMA((NROWS,)),
        pltpu.SemaphoreType.DMA((NROWS,)),
    ],
)
def _scatter_kernel(hw_hbm, eidx_hbm, zeros_hbm, out_hbm,
                    sidx_v, didx_v, rows_v, agg_sh, sem_i, sem_g, sem_s):
    c = lax.axis_index("c")
    s = lax.axis_index("s")
    tbase = (c * NS + s) * ET2

    def idx_refs(j, sl):
        sb = pl.ds(pl.multiple_of(tbase + j * CH, 8), CH)
        db = pl.ds(pl.multiple_of(E + tbase + j * CH, 8), CH)
        return ((eidx_hbm.at[sb], sidx_v.at[sl], sem_i.at[sl]),
                (eidx_hbm.at[db], didx_v.at[sl], sem_i.at[sl]))

    def idx_start(j, sl):
        for tr in idx_refs(j, sl):
            pltpu.async_copy(*tr)

    def idx_wait(j, sl):
        for tr in idx_refs(j, sl):
            pltpu.make_async_copy(*tr).wait()

    def gather_refs(ri, ii):
        return (hw_hbm.at[sidx_v.at[ii]], rows_v.at[ri], sem_g.at[ri])

    def scat_refs(ri, ii):
        return (rows_v.at[ri], agg_sh.at[didx_v.at[ii]], sem_s.at[ri])

    pltpu.sync_copy(zeros_hbm.at[pl.ds(s * RPT, RPT)], agg_sh.at[pl.ds(s * RPT, RPT)])
    plsc.subcore_barrier()

    for jj in range(3):
        idx_start(jj, jj)
    for jj in range(2):
        idx_wait(jj, jj)
        pltpu.async_copy(*gather_refs(jj, jj))

    def step(j, _):
        p = lax.rem(j, NROWS)
        pi = lax.rem(j, NIDX)

        pltpu.make_async_copy(*gather_refs(p, pi)).wait()

        @pl.when(j > 1)
        def _():
            pltpu.make_async_copy(
                *scat_refs(lax.rem(j + NROWS - 2, NROWS),
                           lax.rem(j + NIDX - 2, NIDX))).wait()

        pltpu.async_copy(*scat_refs(p, pi), add=True)

        @pl.when(j + 3 < TCHUNKS2)
        def _():
            idx_start(j + 3, lax.rem(j + 3, NIDX))

        @pl.when(j + 2 < TCHUNKS2)
        def _():
            idx_wait(j + 2, lax.rem(j + 2, NIDX))
            pltpu.async_copy(*gather_refs(lax.rem(j + 2, NROWS), lax.rem(j + 2, NIDX)))
        return 0
    lax.fori_loop(0, TCHUNKS2, step, 0)
    for jj in (TCHUNKS2 - 2, TCHUNKS2 - 1):
        pltpu.make_async_copy(*scat_refs(jj % NROWS, jj % NIDX)).wait()

    plsc.subcore_barrier()
    obase = pl.multiple_of(c * NP + s * RPT, 8)
    pltpu.sync_copy(agg_sh.at[pl.ds(s * RPT, RPT)], out_hbm.at[pl.ds(obase, RPT)])


def _scale_of(deg_ref):
    return lax.rsqrt(jnp.maximum(deg_ref[...], 1.0))


def _f0_body(x_ref, do_ref, w_ref, out_ref):
    out_ref[0:N, :] = lax.dot_general(x_ref[...] * _scale_of(do_ref), w_ref[...],
                                      (((1,), (1,)), ((), ())),
                                      preferred_element_type=jnp.float32)


def _bn_relu(agg_ref, di_ref, g_ref, b_ref):
    a = agg_ref[...]
    h = a[0:N] + a[NP:NP + N]
    h = h * _scale_of(di_ref)
    m = jnp.mean(h, axis=0, keepdims=True)
    d = h - m
    v = jnp.mean(d * d, axis=0, keepdims=True)
    hn = d * lax.rsqrt(v + 1e-5) * g_ref[...] + b_ref[...]
    return jnp.maximum(hn, 0.0)


def _f1_body(agg_ref, do_ref, di_ref, g_ref, b_ref, w_ref, out_ref):
    hr = _bn_relu(agg_ref, di_ref, g_ref, b_ref)
    out_ref[0:N, :] = lax.dot_general(hr * _scale_of(do_ref), w_ref[...],
                                      (((1,), (1,)), ((), ())),
                                      preferred_element_type=jnp.float32)


def _fc_body(agg_ref, di_ref, g_ref, b_ref, wc_ref, bc_ref, out_ref):
    hr = _bn_relu(agg_ref, di_ref, g_ref, b_ref)
    out_ref[...] = lax.dot_general(hr, wc_ref[...],
                                   (((1,), (1,)), ((), ())),
                                   preferred_element_type=jnp.float32) + bc_ref[...]


_f0 = pl.pallas_call(_f0_body, out_shape=jax.ShapeDtypeStruct((NP, D), jnp.float32))
_f1 = pl.pallas_call(_f1_body, out_shape=jax.ShapeDtypeStruct((NP, D), jnp.float32))
_fc = pl.pallas_call(_fc_body, out_shape=jax.ShapeDtypeStruct((N, 40), jnp.float32))


def kernel(x, edge_index, W0, g0, b0, W1, g1, b1, W2, g2, b2, Wc, bc):
    eidx = edge_index.reshape(2 * E)
    zeros1 = jnp.zeros((NP,), jnp.float32)
    zeros128 = jnp.zeros((NP, D), jnp.float32)
    g0r, g1r, g2r = g0.reshape(1, D), g1.reshape(1, D), g2.reshape(1, D)
    b0r, b1r, b2r = b0.reshape(1, D), b1.reshape(1, D), b2.reshape(1, D)
    bcr = bc.reshape(1, 40)

    degs = _deg_kernel(eidx, zeros1, jnp.ones((CH,), jnp.float32))
    do = degs[0:N].reshape(N, 1)
    di = degs[NP:NP + N].reshape(N, 1)

    hw = _f0(x, do, W0)
    agg = _scatter_kernel(hw, eidx, zeros128)
    hw = _f1(agg, do, di, g0r, b0r, W1)
    agg = _scatter_kernel(hw, eidx, zeros128)
    hw = _f1(agg, do, di, g1r, b1r, W2)
    agg = _scatter_kernel(hw, eidx, zeros128)
    return _fc(agg, di, g2r, b2r, Wc, bcr)


# speedup vs baseline: 13.1453x; 1.0346x over previous
import functools

import jax
import jax.numpy as jnp
from jax import lax
from jax.experimental import pallas as pl
from jax.experimental.pallas import tpu as pltpu
from jax.experimental.pallas import tpu_sc as plsc

N = 10000
NP = 10240
E = 320000
D = 128
NC = 2
NS = 16
CH = 80
ET = E // NS
TCHUNKS = ET // CH
ET2 = E // (NC * NS)
TCHUNKS2 = ET2 // CH
RPT = NP // NS

_MESH = plsc.VectorSubcoreMesh(
    core_axis_name="c", subcore_axis_name="s", num_cores=NC, num_subcores=NS)


@functools.partial(
    pl.kernel,
    out_type=jax.ShapeDtypeStruct((NC * NP,), jnp.float32),
    mesh=_MESH,
    scratch_types=[
        pltpu.VMEM((ET,), jnp.int32),
        pltpu.VMEM((3, CH), jnp.int32),
        pltpu.VMEM((CH,), jnp.float32),
        pltpu.VMEM_SHARED((NP,), jnp.float32),
        pltpu.SemaphoreType.DMA((3,)),
    ],
)
def _deg_kernel(eidx_hbm, zeros_hbm, ones_hbm, out_hbm,
                idx_v, didx_v, ones_v, deg_sh, sem_s):
    c = lax.axis_index("c")
    s = lax.axis_index("s")

    pltpu.sync_copy(ones_hbm, ones_v)
    base = pl.multiple_of(c * E + s * ET, 8)
    pltpu.sync_copy(eidx_hbm.at[pl.ds(base, ET)], idx_v)
    pltpu.sync_copy(zeros_hbm.at[pl.ds(s * RPT, RPT)], deg_sh.at[pl.ds(s * RPT, RPT)])
    plsc.subcore_barrier()

    def step(j, _):
        p = lax.rem(j, 3)

        @pl.when(j > 1)
        def _():
            q = lax.rem(j + 1, 3)
            pltpu.make_async_copy(ones_v, deg_sh.at[didx_v.at[q]], sem_s.at[q]).wait()

        for i in range(CH // 16):
            didx_v[p, pl.ds(i * 16, 16)] = idx_v[pl.ds(j * CH + i * 16, 16)]
        pltpu.async_copy(ones_v, deg_sh.at[didx_v.at[p]], sem_s.at[p], add=True)
        return 0
    lax.fori_loop(0, TCHUNKS, step, 0)
    for jj in (TCHUNKS - 2, TCHUNKS - 1):
        pltpu.make_async_copy(ones_v, deg_sh.at[didx_v.at[jj % 3]],
                              sem_s.at[jj % 3]).wait()

    plsc.subcore_barrier()
    obase = pl.multiple_of(c * NP + s * RPT, 8)
    pltpu.sync_copy(deg_sh.at[pl.ds(s * RPT, RPT)], out_hbm.at[pl.ds(obase, RPT)])


NROWS = 3
NDIDX = 4
CHS = 128
ESC = E // NC
NCHS = ESC // CHS
BCT = NCHS // NS
XTRA = NCHS % NS
NPA = 10112
RPTA = NPA // NS


@functools.partial(
    pl.kernel,
    out_type=jax.ShapeDtypeStruct((NC * NPA, D), jnp.float32),
    mesh=_MESH,
    scratch_types=[
        pltpu.VMEM((NROWS, CHS), jnp.int32),
        pltpu.VMEM((NDIDX, CHS), jnp.int32),
        pltpu.VMEM((NROWS, CHS, D), jnp.float32),
        pltpu.VMEM_SHARED((NPA, D), jnp.float32),
        pltpu.SemaphoreType.DMA((NDIDX,)),
        pltpu.SemaphoreType.DMA((NROWS,)),
        pltpu.SemaphoreType.DMA((NROWS,)),
    ],
)
def _scatter_kernel(hw_hbm, eidx_hbm, zeros_hbm, out_hbm,
                    sidx_v, didx_v, rows_v, agg_sh, sem_i, sem_g, sem_s):
    c = lax.axis_index("c")
    s = lax.axis_index("s")
    nchunks = BCT + (s < XTRA).astype(jnp.int32)
    cbase = BCT * s + jnp.minimum(s, XTRA)
    tbase = c * ESC + cbase * CHS

    def idx_refs(j, ss, ds):
        sb = pl.ds(pl.multiple_of(tbase + j * CHS, 8), CHS)
        db = pl.ds(pl.multiple_of(E + tbase + j * CHS, 8), CHS)
        return ((eidx_hbm.at[sb], sidx_v.at[ss], sem_i.at[ds]),
                (eidx_hbm.at[db], didx_v.at[ds], sem_i.at[ds]))

    def idx_start(j, ss, ds):
        for tr in idx_refs(j, ss, ds):
            pltpu.async_copy(*tr)

    def idx_wait(j, ss, ds):
        for tr in idx_refs(j, ss, ds):
            pltpu.make_async_copy(*tr).wait()

    def gather_refs(ri):
        return (hw_hbm.at[sidx_v.at[ri]], rows_v.at[ri], sem_g.at[ri])

    def scat_refs(ri, di):
        return (rows_v.at[ri], agg_sh.at[didx_v.at[di]], sem_s.at[ri])

    pltpu.sync_copy(zeros_hbm.at[pl.ds(s * RPTA, RPTA)],
                    agg_sh.at[pl.ds(s * RPTA, RPTA)])
    plsc.subcore_barrier()

    for jj in range(3):
        idx_start(jj, jj, jj)
    for jj in range(2):
        idx_wait(jj, jj, jj)
        pltpu.async_copy(*gather_refs(jj))

    def step(j, _):
        p3 = lax.rem(j, NROWS)
        p4 = lax.rem(j, NDIDX)

        pltpu.make_async_copy(*gather_refs(p3)).wait()

        @pl.when(j > 0)
        def _():
            pltpu.make_async_copy(
                *scat_refs(lax.rem(j + NROWS - 1, NROWS),
                           lax.rem(j + NDIDX - 1, NDIDX))).wait()

        pltpu.async_copy(*scat_refs(p3, p4), add=True)

        @pl.when(j + 3 < nchunks)
        def _():
            idx_start(j + 3, p3, lax.rem(j + 3, NDIDX))

        @pl.when(j + 2 < nchunks)
        def _():
            r3 = lax.rem(j + 2, NROWS)
            idx_wait(j + 2, r3, lax.rem(j + 2, NDIDX))
            pltpu.async_copy(*gather_refs(r3))
        return 0
    lax.fori_loop(0, nchunks, step, 0)
    pltpu.make_async_copy(
        *scat_refs(lax.rem(nchunks - 1, NROWS),
                   lax.rem(nchunks - 1, NDIDX))).wait()

    plsc.subcore_barrier()
    obase = pl.multiple_of(c * NPA + s * RPTA, 8)
    pltpu.sync_copy(agg_sh.at[pl.ds(s * RPTA, RPTA)], out_hbm.at[pl.ds(obase, RPTA)])


def _scale_of(deg_ref):
    return lax.rsqrt(jnp.maximum(deg_ref[...], 1.0))


def _f0_body(x_ref, do_ref, w_ref, out_ref):
    out_ref[0:N, :] = lax.dot_general(x_ref[...] * _scale_of(do_ref), w_ref[...],
                                      (((1,), (1,)), ((), ())),
                                      preferred_element_type=jnp.float32)


def _bn_relu(agg_ref, di_ref, g_ref, b_ref):
    a = agg_ref[...]
    h = a[0:N] + a[NPA:NPA + N]
    h = h * _scale_of(di_ref)
    m = jnp.mean(h, axis=0, keepdims=True)
    d = h - m
    v = jnp.mean(d * d, axis=0, keepdims=True)
    hn = d * lax.rsqrt(v + 1e-5) * g_ref[...] + b_ref[...]
    return jnp.maximum(hn, 0.0)


def _f1_body(agg_ref, do_ref, di_ref, g_ref, b_ref, w_ref, out_ref):
    hr = _bn_relu(agg_ref, di_ref, g_ref, b_ref)
    out_ref[0:N, :] = lax.dot_general(hr * _scale_of(do_ref), w_ref[...],
                                      (((1,), (1,)), ((), ())),
                                      preferred_element_type=jnp.float32)


def _fc_body(agg_ref, di_ref, g_ref, b_ref, wc_ref, bc_ref, out_ref):
    hr = _bn_relu(agg_ref, di_ref, g_ref, b_ref)
    out_ref[...] = lax.dot_general(hr, wc_ref[...],
                                   (((1,), (1,)), ((), ())),
                                   preferred_element_type=jnp.float32) + bc_ref[...]


_f0 = pl.pallas_call(_f0_body, out_shape=jax.ShapeDtypeStruct((NP, D), jnp.float32))
_f1 = pl.pallas_call(_f1_body, out_shape=jax.ShapeDtypeStruct((NP, D), jnp.float32))
_fc = pl.pallas_call(_fc_body, out_shape=jax.ShapeDtypeStruct((N, 40), jnp.float32))


def kernel(x, edge_index, W0, g0, b0, W1, g1, b1, W2, g2, b2, Wc, bc):
    eidx = edge_index.reshape(2 * E)
    zeros1 = jnp.zeros((NP,), jnp.float32)
    zeros128 = jnp.zeros((NPA, D), jnp.float32)
    g0r, g1r, g2r = g0.reshape(1, D), g1.reshape(1, D), g2.reshape(1, D)
    b0r, b1r, b2r = b0.reshape(1, D), b1.reshape(1, D), b2.reshape(1, D)
    bcr = bc.reshape(1, 40)

    degs = _deg_kernel(eidx, zeros1, jnp.ones((CH,), jnp.float32))
    do = degs[0:N].reshape(N, 1)
    di = degs[NP:NP + N].reshape(N, 1)

    hw = _f0(x, do, W0)
    agg = _scatter_kernel(hw, eidx, zeros128)
    hw = _f1(agg, do, di, g0r, b0r, W1)
    agg = _scatter_kernel(hw, eidx, zeros128)
    hw = _f1(agg, do, di, g1r, b1r, W2)
    agg = _scatter_kernel(hw, eidx, zeros128)
    return _fc(agg, di, g2r, b2r, Wc, bcr)
